# Initial kernel scaffold; baseline (speedup 1.0000x reference)
import jax
import jax.numpy as jnp
from jax.experimental import pallas as pl


def kernel(x, G2_edge_attr, G1_edge_attr_matrix, G3_edge_index, G3_edge_attr, W_l1, W_r1, b1, W_l2, W_r2, b2):
    raise NotImplementedError("write your pallas kernel here")


# speedup vs baseline: 4.4821x; 4.4821x over previous
import jax
import jax.numpy as jnp
from jax import lax
from jax.experimental import pallas as pl
from jax.experimental.pallas import tpu as pltpu
from jax.experimental.pallas import tpu_sc as plsc

N = 10000
E = 160000
D = 256
H = 256
C = 64

NC = 2
NS = 16
QW = H // 4
DW = 16

BM = 1000
RPT = 624
TAIL = N - NS * RPT
B1 = 80
K1 = (E // NS) // B1
B2 = 40
K2 = (E // (NC * NS)) // B2

_F32 = jnp.float32


def _fill(ref, rows, cols, value):
    def row(i, carry):
        def col(k, carry2):
            ref[i, pl.ds(k * 16, 16)] = jnp.full((16,), value, _F32)
            return carry2
        lax.fori_loop(0, cols // 16, col, 0)
        return carry
    lax.fori_loop(0, rows, row, 0)


def _zero_slices(zbuf, brows, acc, base):
    nfull = RPT // brows
    rem = RPT - nfull * brows
    def blk(k, carry):
        pltpu.sync_copy(zbuf, acc.at[pl.ds(base + k * brows, brows)])
        return carry
    lax.fori_loop(0, nfull, blk, 0)
    if rem:
        pltpu.sync_copy(zbuf.at[pl.ds(0, rem)],
                        acc.at[pl.ds(base + nfull * brows, rem)])


def _sc_l1(p1q0, p1q1, p1q2, p1q3, srcm, dstm,
           agg_out, deg_out,
           src_v, dst_v, rows_v, ones_v, zdeg_v, acc_s, deg_s, sem):
    c = lax.axis_index("c")
    s = lax.axis_index("s")

    @pl.when(c == 0)
    def _():
        _fill(ones_v, B1, DW, 1.0)
        _fill(zdeg_v, B1, DW, 0.0)
        _zero_slices(zdeg_v, B1, deg_s, s * RPT)

    @pl.when(jnp.logical_and(c == 0, s == 0))
    def _():
        pltpu.sync_copy(zdeg_v.at[pl.ds(0, TAIL)], deg_s.at[pl.ds(NS * RPT, TAIL)])

    pltpu.sync_copy(srcm.at[s], src_v)
    pltpu.sync_copy(dstm.at[s], dst_v)

    def one_pass(p_ref, q, add_deg):
        _fill(rows_v, B1, QW, 0.0)
        _zero_slices(rows_v, B1, acc_s, s * RPT)

        @pl.when(s == 0)
        def _():
            pltpu.sync_copy(rows_v.at[pl.ds(0, TAIL)],
                            acc_s.at[pl.ds(NS * RPT, TAIL)])

        plsc.subcore_barrier()

        def chunk(j, carry):
            pltpu.async_copy(p_ref.at[src_v.at[j]], rows_v, sem).wait()
            pltpu.sync_copy(rows_v, acc_s.at[dst_v.at[j]], add=True)
            if add_deg:
                pltpu.sync_copy(ones_v, deg_s.at[dst_v.at[j]], add=True)
            return carry
        lax.fori_loop(0, K1, chunk, 0)

        plsc.subcore_barrier()
        pltpu.sync_copy(acc_s.at[pl.ds(s * RPT, RPT)],
                        agg_out.at[pl.ds(q * N + s * RPT, RPT)])

        @pl.when(s == 0)
        def _():
            pltpu.sync_copy(acc_s.at[pl.ds(NS * RPT, TAIL)],
                            agg_out.at[pl.ds(q * N + NS * RPT, TAIL)])

        plsc.subcore_barrier()

    @pl.when(c == 0)
    def _():
        one_pass(p1q0, 0, True)
        one_pass(p1q1, 1, False)

    @pl.when(c == 1)
    def _():
        one_pass(p1q2, 2, False)
        one_pass(p1q3, 3, False)

    @pl.when(c == 0)
    def _():
        pltpu.sync_copy(deg_s.at[pl.ds(s * RPT, RPT)],
                        deg_out.at[pl.ds(s * RPT, RPT)])

    @pl.when(jnp.logical_and(c == 0, s == 0))
    def _():
        pltpu.sync_copy(deg_s.at[pl.ds(NS * RPT, TAIL)],
                        deg_out.at[pl.ds(NS * RPT, TAIL)])


def _sc_l2(p2, srcm, dstm,
           agg_out,
           src_v, dst_v, rows_v, acc_s, sem):
    c = lax.axis_index("c")
    s = lax.axis_index("s")
    wid = s * NC + c
    _fill(rows_v, B2, C, 0.0)
    _zero_slices(rows_v, B2, acc_s, s * RPT)

    @pl.when(s == 0)
    def _():
        pltpu.sync_copy(rows_v.at[pl.ds(0, TAIL)], acc_s.at[pl.ds(NS * RPT, TAIL)])

    pltpu.sync_copy(srcm.at[wid], src_v)
    pltpu.sync_copy(dstm.at[wid], dst_v)
    plsc.subcore_barrier()

    def chunk(j, carry):
        pltpu.async_copy(p2.at[src_v.at[j]], rows_v, sem).wait()
        pltpu.sync_copy(rows_v, acc_s.at[dst_v.at[j]], add=True)
        return carry
    lax.fori_loop(0, K2, chunk, 0)

    plsc.subcore_barrier()
    pltpu.sync_copy(acc_s.at[pl.ds(s * RPT, RPT)],
                    agg_out.at[pl.ds(c * N + s * RPT, RPT)])

    @pl.when(s == 0)
    def _():
        pltpu.sync_copy(acc_s.at[pl.ds(NS * RPT, TAIL)],
                        agg_out.at[pl.ds(c * N + NS * RPT, TAIL)])


def _tc1(x_ref, wl_ref, wr_ref, q0_ref, q1_ref, q2_ref, q3_ref, xr_ref):
    xb = x_ref[...]
    p = jnp.dot(xb, wl_ref[...], preferred_element_type=_F32)
    q0_ref[...] = p[:, 0 * QW:1 * QW]
    q1_ref[...] = p[:, 1 * QW:2 * QW]
    q2_ref[...] = p[:, 2 * QW:3 * QW]
    q3_ref[...] = p[:, 3 * QW:4 * QW]
    xr_ref[...] = jnp.dot(xb, wr_ref[...], preferred_element_type=_F32)


def _tc2(a_ref, b_ref, c_ref, d_ref, deg_ref, xr_ref, b1_ref, wl2_ref, wr2_ref,
         b2_ref, p2_ref, hr_ref):
    agg = jnp.concatenate([a_ref[...], b_ref[...], c_ref[...], d_ref[...]],
                          axis=1)
    deg = jnp.maximum(deg_ref[...][:, 0:1], 1.0)
    h = jnp.maximum(agg / deg + xr_ref[...] + b1_ref[...], 0.0)
    p2_ref[...] = jnp.dot(h, wl2_ref[...], preferred_element_type=_F32)
    hr_ref[...] = jnp.dot(h, wr2_ref[...], preferred_element_type=_F32) + b2_ref[...]


def _tc3(a_ref, b_ref, deg_ref, hr_ref, out_ref):
    deg = jnp.maximum(deg_ref[...][:, 0:1], 1.0)
    v = (a_ref[...] + b_ref[...]) / deg + hr_ref[...]
    m = jnp.max(v, axis=1, keepdims=True)
    z = v - m
    lse = jnp.log(jnp.sum(jnp.exp(z), axis=1, keepdims=True))
    out_ref[...] = z - lse


def kernel(x, G2_edge_attr, G1_edge_attr_matrix, G3_edge_index, G3_edge_attr,
           W_l1, W_r1, b1, W_l2, W_r2, b2):
    src = G3_edge_index[0]
    dst = G3_edge_index[1]
    src1 = src.reshape(NS, K1, B1)
    dst1 = dst.reshape(NS, K1, B1)
    src2 = src.reshape(NC * NS, K2, B2)
    dst2 = dst.reshape(NC * NS, K2, B2)
    b1r = b1.reshape(1, H)
    b2r = b2.reshape(1, C)

    grid = (N // BM,)
    full = lambda i: (0, 0)
    rows = lambda i: (i, 0)
    rows_hi = lambda i: (N // BM + i, 0)

    p1q0, p1q1, p1q2, p1q3, xr1 = pl.pallas_call(
        _tc1,
        grid=grid,
        in_specs=[pl.BlockSpec((BM, D), rows),
                  pl.BlockSpec((D, H), full),
                  pl.BlockSpec((D, H), full)],
        out_specs=[pl.BlockSpec((BM, QW), rows),
                   pl.BlockSpec((BM, QW), rows),
                   pl.BlockSpec((BM, QW), rows),
                   pl.BlockSpec((BM, QW), rows),
                   pl.BlockSpec((BM, H), rows)],
        out_shape=[jax.ShapeDtypeStruct((N, QW), _F32),
                   jax.ShapeDtypeStruct((N, QW), _F32),
                   jax.ShapeDtypeStruct((N, QW), _F32),
                   jax.ShapeDtypeStruct((N, QW), _F32),
                   jax.ShapeDtypeStruct((N, H), _F32)],
    )(x, W_l1, W_r1)

    mesh = plsc.VectorSubcoreMesh(core_axis_name="c", subcore_axis_name="s")
    sc_params = pltpu.CompilerParams(use_tc_tiling_on_sc=False)
    agg1, deg8 = pl.kernel(
        _sc_l1,
        compiler_params=sc_params,
        out_type=(jax.ShapeDtypeStruct((4 * N, QW), _F32),
                  jax.ShapeDtypeStruct((N, DW), _F32)),
        mesh=mesh,
        scratch_types=(
            pltpu.VMEM((K1, B1), jnp.int32),
            pltpu.VMEM((K1, B1), jnp.int32),
            pltpu.VMEM((B1, QW), _F32),
            pltpu.VMEM((B1, DW), _F32),
            pltpu.VMEM((B1, DW), _F32),
            pltpu.VMEM_SHARED((N, QW), _F32),
            pltpu.VMEM_SHARED((N, DW), _F32),
            pltpu.SemaphoreType.DMA,
        ),
    )(p1q0, p1q1, p1q2, p1q3, src1, dst1)

    qrows = [lambda i, q=q: (q * (N // BM) + i, 0) for q in range(4)]
    p2, hr2 = pl.pallas_call(
        _tc2,
        grid=grid,
        in_specs=[pl.BlockSpec((BM, QW), qrows[0]),
                  pl.BlockSpec((BM, QW), qrows[1]),
                  pl.BlockSpec((BM, QW), qrows[2]),
                  pl.BlockSpec((BM, QW), qrows[3]),
                  pl.BlockSpec((BM, DW), rows),
                  pl.BlockSpec((BM, H), rows),
                  pl.BlockSpec((1, H), full),
                  pl.BlockSpec((H, C), full),
                  pl.BlockSpec((H, C), full),
                  pl.BlockSpec((1, C), full)],
        out_specs=[pl.BlockSpec((BM, C), rows),
                   pl.BlockSpec((BM, C), rows)],
        out_shape=[jax.ShapeDtypeStruct((N, C), _F32),
                   jax.ShapeDtypeStruct((N, C), _F32)],
    )(agg1, agg1, agg1, agg1, deg8, xr1, b1r, W_l2, W_r2, b2r)

    agg2 = pl.kernel(
        _sc_l2,
        compiler_params=sc_params,
        out_type=jax.ShapeDtypeStruct((2 * N, C), _F32),
        mesh=mesh,
        scratch_types=(
            pltpu.VMEM((K2, B2), jnp.int32),
            pltpu.VMEM((K2, B2), jnp.int32),
            pltpu.VMEM((B2, C), _F32),
            pltpu.VMEM_SHARED((N, C), _F32),
            pltpu.SemaphoreType.DMA,
        ),
    )(p2, src2, dst2)

    out = pl.pallas_call(
        _tc3,
        grid=grid,
        in_specs=[pl.BlockSpec((BM, C), rows),
                  pl.BlockSpec((BM, C), rows_hi),
                  pl.BlockSpec((BM, DW), rows),
                  pl.BlockSpec((BM, C), rows)],
        out_specs=pl.BlockSpec((BM, C), rows),
        out_shape=jax.ShapeDtypeStruct((N, C), _F32),
    )(agg2, agg2, deg8, hr2)

    return out


# speedup vs baseline: 7.9164x; 1.7662x over previous
import jax
import jax.numpy as jnp
from jax import lax
from jax.experimental import pallas as pl
from jax.experimental.pallas import tpu as pltpu
from jax.experimental.pallas import tpu_sc as plsc

N = 10000
E = 160000
D = 256
H = 256
C = 64

NC = 2
NS = 16
QW = H // 4
DW = 16

BM = 1000
RPT = 624
TAIL = N - NS * RPT
B1 = 125
K1 = (E // NS) // B1
B2 = 125
K2 = (E // (NC * NS)) // B2

_F32 = jnp.float32


def _fill(ref, rows, cols, value):
    def row(i, carry):
        def col(k, carry2):
            ref[i, pl.ds(k * 16, 16)] = jnp.full((16,), value, _F32)
            return carry2
        lax.fori_loop(0, cols // 16, col, 0)
        return carry
    lax.fori_loop(0, rows, row, 0)


def _zero_slices(zbuf, brows, acc, base):
    nfull = RPT // brows
    rem = RPT - nfull * brows
    def blk(k, carry):
        pltpu.sync_copy(zbuf, acc.at[pl.ds(base + k * brows, brows)])
        return carry
    lax.fori_loop(0, nfull, blk, 0)
    if rem:
        pltpu.sync_copy(zbuf.at[pl.ds(0, rem)],
                        acc.at[pl.ds(base + nfull * brows, rem)])


def _edge_loop(p_ref, src_v, dst_v, acc_s, K, bufs, sems, extra=None):
    pltpu.async_copy(p_ref.at[src_v.at[0]], bufs[0], sems[0])
    pltpu.async_copy(p_ref.at[src_v.at[1]], bufs[1], sems[1])

    def step(t, carry):
        j0 = t * 2
        for b in (0, 1):
            j = j0 + b
            pltpu.make_async_copy(p_ref.at[src_v.at[j]], bufs[b], sems[b]).wait()
            pltpu.sync_copy(bufs[b], acc_s.at[dst_v.at[j]], add=True)
            if extra is not None:
                ones_v, deg_s = extra
                pltpu.sync_copy(ones_v, deg_s.at[dst_v.at[j]], add=True)
            nxt = j + 2

            @pl.when(nxt < K)
            def _():
                pltpu.async_copy(p_ref.at[src_v.at[nxt]], bufs[b], sems[b])
        return carry
    lax.fori_loop(0, K // 2, step, 0)


def _sc_l1(p1q0, p1q1, p1q2, p1q3, srcm, dstm,
           agg_out, deg_out,
           src_v, dst_v, rows_v, rows_w, ones_v, zdeg_v, acc_s, deg_s,
           sem_a, sem_b):
    c = lax.axis_index("c")
    s = lax.axis_index("s")

    @pl.when(c == 0)
    def _():
        _fill(ones_v, B1, DW, 1.0)
        _fill(zdeg_v, B1, DW, 0.0)
        _zero_slices(zdeg_v, B1, deg_s, s * RPT)

    @pl.when(jnp.logical_and(c == 0, s == 0))
    def _():
        pltpu.sync_copy(zdeg_v.at[pl.ds(0, TAIL)], deg_s.at[pl.ds(NS * RPT, TAIL)])

    pltpu.sync_copy(srcm.at[s], src_v)
    pltpu.sync_copy(dstm.at[s], dst_v)

    def one_pass(p_ref, q, add_deg):
        _fill(rows_v, B1, QW, 0.0)
        _zero_slices(rows_v, B1, acc_s, s * RPT)

        @pl.when(s == 0)
        def _():
            pltpu.sync_copy(rows_v.at[pl.ds(0, TAIL)],
                            acc_s.at[pl.ds(NS * RPT, TAIL)])

        plsc.subcore_barrier()
        _edge_loop(p_ref, src_v, dst_v, acc_s, K1, (rows_v, rows_w),
                   (sem_a, sem_b),
                   extra=(ones_v, deg_s) if add_deg else None)

        plsc.subcore_barrier()
        pltpu.sync_copy(acc_s.at[pl.ds(s * RPT, RPT)],
                        agg_out.at[pl.ds(q * N + s * RPT, RPT)])

        @pl.when(s == 0)
        def _():
            pltpu.sync_copy(acc_s.at[pl.ds(NS * RPT, TAIL)],
                            agg_out.at[pl.ds(q * N + NS * RPT, TAIL)])

        plsc.subcore_barrier()

    @pl.when(c == 0)
    def _():
        one_pass(p1q0, 0, True)
        one_pass(p1q1, 1, False)

    @pl.when(c == 1)
    def _():
        one_pass(p1q2, 2, False)
        one_pass(p1q3, 3, False)

    @pl.when(c == 0)
    def _():
        pltpu.sync_copy(deg_s.at[pl.ds(s * RPT, RPT)],
                        deg_out.at[pl.ds(s * RPT, RPT)])

    @pl.when(jnp.logical_and(c == 0, s == 0))
    def _():
        pltpu.sync_copy(deg_s.at[pl.ds(NS * RPT, TAIL)],
                        deg_out.at[pl.ds(NS * RPT, TAIL)])


def _sc_l2(p2, srcm, dstm,
           agg_out,
           src_v, dst_v, rows_v, rows_w, acc_s, sem_a, sem_b):
    c = lax.axis_index("c")
    s = lax.axis_index("s")
    wid = s * NC + c
    _fill(rows_v, B2, C, 0.0)
    _zero_slices(rows_v, B2, acc_s, s * RPT)

    @pl.when(s == 0)
    def _():
        pltpu.sync_copy(rows_v.at[pl.ds(0, TAIL)], acc_s.at[pl.ds(NS * RPT, TAIL)])

    pltpu.sync_copy(srcm.at[wid], src_v)
    pltpu.sync_copy(dstm.at[wid], dst_v)
    plsc.subcore_barrier()
    _edge_loop(p2, src_v, dst_v, acc_s, K2, (rows_v, rows_w),
               (sem_a, sem_b))

    plsc.subcore_barrier()
    pltpu.sync_copy(acc_s.at[pl.ds(s * RPT, RPT)],
                    agg_out.at[pl.ds(c * N + s * RPT, RPT)])

    @pl.when(s == 0)
    def _():
        pltpu.sync_copy(acc_s.at[pl.ds(NS * RPT, TAIL)],
                        agg_out.at[pl.ds(c * N + NS * RPT, TAIL)])


def _tc1(x_ref, wl_ref, wr_ref, q0_ref, q1_ref, q2_ref, q3_ref, xr_ref):
    xb = x_ref[...]
    p = jnp.dot(xb, wl_ref[...], preferred_element_type=_F32)
    q0_ref[...] = p[:, 0 * QW:1 * QW]
    q1_ref[...] = p[:, 1 * QW:2 * QW]
    q2_ref[...] = p[:, 2 * QW:3 * QW]
    q3_ref[...] = p[:, 3 * QW:4 * QW]
    xr_ref[...] = jnp.dot(xb, wr_ref[...], preferred_element_type=_F32)


def _tc2(a_ref, b_ref, c_ref, d_ref, deg_ref, xr_ref, b1_ref, wl2_ref, wr2_ref,
         b2_ref, p2_ref, hr_ref):
    agg = jnp.concatenate([a_ref[...], b_ref[...], c_ref[...], d_ref[...]],
                          axis=1)
    deg = jnp.maximum(deg_ref[...][:, 0:1], 1.0)
    h = jnp.maximum(agg / deg + xr_ref[...] + b1_ref[...], 0.0)
    p2_ref[...] = jnp.dot(h, wl2_ref[...], preferred_element_type=_F32)
    hr_ref[...] = jnp.dot(h, wr2_ref[...], preferred_element_type=_F32) + b2_ref[...]


def _tc3(a_ref, b_ref, deg_ref, hr_ref, out_ref):
    deg = jnp.maximum(deg_ref[...][:, 0:1], 1.0)
    v = (a_ref[...] + b_ref[...]) / deg + hr_ref[...]
    m = jnp.max(v, axis=1, keepdims=True)
    z = v - m
    lse = jnp.log(jnp.sum(jnp.exp(z), axis=1, keepdims=True))
    out_ref[...] = z - lse


def kernel(x, G2_edge_attr, G1_edge_attr_matrix, G3_edge_index, G3_edge_attr,
           W_l1, W_r1, b1, W_l2, W_r2, b2):
    src = G3_edge_index[0]
    dst = G3_edge_index[1]
    src1 = src.reshape(NS, K1, B1)
    dst1 = dst.reshape(NS, K1, B1)
    src2 = src.reshape(NC * NS, K2, B2)
    dst2 = dst.reshape(NC * NS, K2, B2)
    b1r = b1.reshape(1, H)
    b2r = b2.reshape(1, C)

    grid = (N // BM,)
    full = lambda i: (0, 0)
    rows = lambda i: (i, 0)
    rows_hi = lambda i: (N // BM + i, 0)

    p1q0, p1q1, p1q2, p1q3, xr1 = pl.pallas_call(
        _tc1,
        grid=grid,
        in_specs=[pl.BlockSpec((BM, D), rows),
                  pl.BlockSpec((D, H), full),
                  pl.BlockSpec((D, H), full)],
        out_specs=[pl.BlockSpec((BM, QW), rows),
                   pl.BlockSpec((BM, QW), rows),
                   pl.BlockSpec((BM, QW), rows),
                   pl.BlockSpec((BM, QW), rows),
                   pl.BlockSpec((BM, H), rows)],
        out_shape=[jax.ShapeDtypeStruct((N, QW), _F32),
                   jax.ShapeDtypeStruct((N, QW), _F32),
                   jax.ShapeDtypeStruct((N, QW), _F32),
                   jax.ShapeDtypeStruct((N, QW), _F32),
                   jax.ShapeDtypeStruct((N, H), _F32)],
    )(x, W_l1, W_r1)

    mesh = plsc.VectorSubcoreMesh(core_axis_name="c", subcore_axis_name="s")
    sc_params = pltpu.CompilerParams(use_tc_tiling_on_sc=False)
    agg1, deg8 = pl.kernel(
        _sc_l1,
        compiler_params=sc_params,
        out_type=(jax.ShapeDtypeStruct((4 * N, QW), _F32),
                  jax.ShapeDtypeStruct((N, DW), _F32)),
        mesh=mesh,
        scratch_types=(
            pltpu.VMEM((K1, B1), jnp.int32),
            pltpu.VMEM((K1, B1), jnp.int32),
            pltpu.VMEM((B1, QW), _F32),
            pltpu.VMEM((B1, QW), _F32),
            pltpu.VMEM((B1, DW), _F32),
            pltpu.VMEM((B1, DW), _F32),
            pltpu.VMEM_SHARED((N, QW), _F32),
            pltpu.VMEM_SHARED((N, DW), _F32),
            pltpu.SemaphoreType.DMA,
            pltpu.SemaphoreType.DMA,
        ),
    )(p1q0, p1q1, p1q2, p1q3, src1, dst1)

    qrows = [lambda i, q=q: (q * (N // BM) + i, 0) for q in range(4)]
    p2, hr2 = pl.pallas_call(
        _tc2,
        grid=grid,
        in_specs=[pl.BlockSpec((BM, QW), qrows[0]),
                  pl.BlockSpec((BM, QW), qrows[1]),
                  pl.BlockSpec((BM, QW), qrows[2]),
                  pl.BlockSpec((BM, QW), qrows[3]),
                  pl.BlockSpec((BM, DW), rows),
                  pl.BlockSpec((BM, H), rows),
                  pl.BlockSpec((1, H), full),
                  pl.BlockSpec((H, C), full),
                  pl.BlockSpec((H, C), full),
                  pl.BlockSpec((1, C), full)],
        out_specs=[pl.BlockSpec((BM, C), rows),
                   pl.BlockSpec((BM, C), rows)],
        out_shape=[jax.ShapeDtypeStruct((N, C), _F32),
                   jax.ShapeDtypeStruct((N, C), _F32)],
    )(agg1, agg1, agg1, agg1, deg8, xr1, b1r, W_l2, W_r2, b2r)

    agg2 = pl.kernel(
        _sc_l2,
        compiler_params=sc_params,
        out_type=jax.ShapeDtypeStruct((2 * N, C), _F32),
        mesh=mesh,
        scratch_types=(
            pltpu.VMEM((K2, B2), jnp.int32),
            pltpu.VMEM((K2, B2), jnp.int32),
            pltpu.VMEM((B2, C), _F32),
            pltpu.VMEM((B2, C), _F32),
            pltpu.VMEM_SHARED((N, C), _F32),
            pltpu.SemaphoreType.DMA,
            pltpu.SemaphoreType.DMA,
        ),
    )(p2, src2, dst2)

    out = pl.pallas_call(
        _tc3,
        grid=grid,
        in_specs=[pl.BlockSpec((BM, C), rows),
                  pl.BlockSpec((BM, C), rows_hi),
                  pl.BlockSpec((BM, DW), rows),
                  pl.BlockSpec((BM, C), rows)],
        out_specs=pl.BlockSpec((BM, C), rows),
        out_shape=jax.ShapeDtypeStruct((N, C), _F32),
    )(agg2, agg2, deg8, hr2)

    return out


# speedup vs baseline: 8.8999x; 1.1242x over previous
import jax
import jax.numpy as jnp
from jax import lax
from jax.experimental import pallas as pl
from jax.experimental.pallas import tpu as pltpu
from jax.experimental.pallas import tpu_sc as plsc

N = 10000
E = 160000
D = 256
H = 256
C = 64

NC = 2
NS = 16
QW = H // 4
DW = 16

BM = 1000
RPT = 624
TAIL = N - NS * RPT
B = 125
K1 = (E // NS) // B
K2 = (E // (NC * NS)) // B

_F32 = jnp.float32


def _fill(ref, rows, cols, value):
    def row(i, carry):
        def col(k, carry2):
            ref[i, pl.ds(k * 16, 16)] = jnp.full((16,), value, _F32)
            return carry2
        lax.fori_loop(0, cols // 16, col, 0)
        return carry
    lax.fori_loop(0, rows, row, 0)


def _zero_slices(zbuf, brows, acc, base):
    nfull = RPT // brows
    rem = RPT - nfull * brows
    def blk(k, carry):
        pltpu.sync_copy(zbuf, acc.at[pl.ds(base + k * brows, brows)])
        return carry
    lax.fori_loop(0, nfull, blk, 0)
    if rem:
        pltpu.sync_copy(zbuf.at[pl.ds(0, rem)],
                        acc.at[pl.ds(base + nfull * brows, rem)])


def _edge_loop(p_ref, src_v, dst_v, acc_s, K, bufs, gsems, ssems, deg=None):
    nb = 4
    pltpu.async_copy(p_ref.at[src_v.at[0]], bufs[0], gsems[0])
    pltpu.async_copy(p_ref.at[src_v.at[1]], bufs[1], gsems[1])

    def step(t, carry):
        j0 = t * nb
        for b in range(nb):
            j = j0 + b
            bn = (b + 2) % nb
            nxt = j + 2

            @pl.when(nxt < K)
            def _():
                @pl.when(nxt >= nb)
                def _():
                    pltpu.make_async_copy(bufs[bn], acc_s.at[dst_v.at[j]],
                                          ssems[bn]).wait()
                pltpu.async_copy(p_ref.at[src_v.at[nxt]], bufs[bn], gsems[bn])

            pltpu.make_async_copy(p_ref.at[src_v.at[j]], bufs[b],
                                  gsems[b]).wait()
            pltpu.async_copy(bufs[b], acc_s.at[dst_v.at[j]], ssems[b],
                             add=True)
            if deg is not None:
                ones_v, deg_s, dsem = deg
                pltpu.async_copy(ones_v, deg_s.at[dst_v.at[j]], dsem, add=True)
        return carry
    lax.fori_loop(0, K // nb, step, 0)

    for b in range(nb):
        pltpu.make_async_copy(bufs[b], acc_s.at[dst_v.at[0]], ssems[b]).wait()
    if deg is not None:
        ones_v, deg_s, dsem = deg

        def drain(j, carry):
            pltpu.make_async_copy(ones_v, deg_s.at[dst_v.at[0]], dsem).wait()
            return carry
        lax.fori_loop(0, K, drain, 0)


def _sc_l1(p1q0, p1q1, p1q2, p1q3, ei,
           agg_out, deg_out,
           src_v, dst_v, bf0, bf1, bf2, bf3, ones_v, zdeg_v, acc_s, deg_s,
           g0, g1, g2, g3, s0, s1, s2, s3, dsem):
    c = lax.axis_index("c")
    s = lax.axis_index("s")
    bufs = (bf0, bf1, bf2, bf3)
    gsems = (g0, g1, g2, g3)
    ssems = (s0, s1, s2, s3)

    @pl.when(c == 0)
    def _():
        _fill(ones_v, B, DW, 1.0)
        _fill(zdeg_v, B, DW, 0.0)
        _zero_slices(zdeg_v, B, deg_s, s * RPT)

    @pl.when(jnp.logical_and(c == 0, s == 0))
    def _():
        pltpu.sync_copy(zdeg_v.at[pl.ds(0, TAIL)],
                        deg_s.at[pl.ds(NS * RPT, TAIL)])

    pltpu.sync_copy(ei.at[0, pl.ds(s * K1, K1)], src_v)
    pltpu.sync_copy(ei.at[1, pl.ds(s * K1, K1)], dst_v)

    def one_pass(p_ref, q, add_deg):
        _fill(bf0, B, QW, 0.0)
        _zero_slices(bf0, B, acc_s, s * RPT)

        @pl.when(s == 0)
        def _():
            pltpu.sync_copy(bf0.at[pl.ds(0, TAIL)],
                            acc_s.at[pl.ds(NS * RPT, TAIL)])

        plsc.subcore_barrier()
        _edge_loop(p_ref, src_v, dst_v, acc_s, K1, bufs, gsems, ssems,
                   deg=(ones_v, deg_s, dsem) if add_deg else None)

        plsc.subcore_barrier()
        pltpu.sync_copy(acc_s.at[pl.ds(s * RPT, RPT)],
                        agg_out.at[pl.ds(q * N + s * RPT, RPT)])

        @pl.when(s == 0)
        def _():
            pltpu.sync_copy(acc_s.at[pl.ds(NS * RPT, TAIL)],
                            agg_out.at[pl.ds(q * N + NS * RPT, TAIL)])

        plsc.subcore_barrier()

    @pl.when(c == 0)
    def _():
        one_pass(p1q0, 0, True)
        one_pass(p1q1, 1, False)

    @pl.when(c == 1)
    def _():
        one_pass(p1q2, 2, False)
        one_pass(p1q3, 3, False)

    @pl.when(c == 0)
    def _():
        pltpu.sync_copy(deg_s.at[pl.ds(s * RPT, RPT)],
                        deg_out.at[pl.ds(s * RPT, RPT)])

    @pl.when(jnp.logical_and(c == 0, s == 0))
    def _():
        pltpu.sync_copy(deg_s.at[pl.ds(NS * RPT, TAIL)],
                        deg_out.at[pl.ds(NS * RPT, TAIL)])


def _sc_l2(p2, ei,
           agg_out,
           src_v, dst_v, bf0, bf1, bf2, bf3, acc_s,
           g0, g1, g2, g3, s0, s1, s2, s3):
    c = lax.axis_index("c")
    s = lax.axis_index("s")
    wid = s * NC + c
    bufs = (bf0, bf1, bf2, bf3)
    _fill(bf0, B, C, 0.0)
    _zero_slices(bf0, B, acc_s, s * RPT)

    @pl.when(s == 0)
    def _():
        pltpu.sync_copy(bf0.at[pl.ds(0, TAIL)],
                        acc_s.at[pl.ds(NS * RPT, TAIL)])

    pltpu.sync_copy(ei.at[0, pl.ds(wid * K2, K2)], src_v)
    pltpu.sync_copy(ei.at[1, pl.ds(wid * K2, K2)], dst_v)
    plsc.subcore_barrier()
    _edge_loop(p2, src_v, dst_v, acc_s, K2, bufs,
               (g0, g1, g2, g3), (s0, s1, s2, s3))

    plsc.subcore_barrier()
    pltpu.sync_copy(acc_s.at[pl.ds(s * RPT, RPT)],
                    agg_out.at[pl.ds(c * N + s * RPT, RPT)])

    @pl.when(s == 0)
    def _():
        pltpu.sync_copy(acc_s.at[pl.ds(NS * RPT, TAIL)],
                        agg_out.at[pl.ds(c * N + NS * RPT, TAIL)])


def _tc1(x_ref, wl_ref, wr_ref, q0_ref, q1_ref, q2_ref, q3_ref, xr_ref):
    xb = x_ref[...]
    p = jnp.dot(xb, wl_ref[...], preferred_element_type=_F32)
    q0_ref[...] = p[:, 0 * QW:1 * QW]
    q1_ref[...] = p[:, 1 * QW:2 * QW]
    q2_ref[...] = p[:, 2 * QW:3 * QW]
    q3_ref[...] = p[:, 3 * QW:4 * QW]
    xr_ref[...] = jnp.dot(xb, wr_ref[...], preferred_element_type=_F32)


def _tc2(a_ref, b_ref, c_ref, d_ref, deg_ref, xr_ref, b1_ref, wl2_ref, wr2_ref,
         b2_ref, p2_ref, hr_ref):
    agg = jnp.concatenate([a_ref[...], b_ref[...], c_ref[...], d_ref[...]],
                          axis=1)
    deg = jnp.maximum(deg_ref[...][:, 0:1], 1.0)
    h = jnp.maximum(agg / deg + xr_ref[...] + b1_ref[...], 0.0)
    p2_ref[...] = jnp.dot(h, wl2_ref[...], preferred_element_type=_F32)
    hr_ref[...] = jnp.dot(h, wr2_ref[...], preferred_element_type=_F32) + b2_ref[...]


def _tc3(a_ref, b_ref, deg_ref, hr_ref, out_ref):
    deg = jnp.maximum(deg_ref[...][:, 0:1], 1.0)
    v = (a_ref[...] + b_ref[...]) / deg + hr_ref[...]
    m = jnp.max(v, axis=1, keepdims=True)
    z = v - m
    lse = jnp.log(jnp.sum(jnp.exp(z), axis=1, keepdims=True))
    out_ref[...] = z - lse


def kernel(x, G2_edge_attr, G1_edge_attr_matrix, G3_edge_index, G3_edge_attr,
           W_l1, W_r1, b1, W_l2, W_r2, b2):
    ei = G3_edge_index.reshape(2, E // B, B)
    b1r = b1.reshape(1, H)
    b2r = b2.reshape(1, C)

    grid = (N // BM,)
    full = lambda i: (0, 0)
    rows = lambda i: (i, 0)
    rows_hi = lambda i: (N // BM + i, 0)

    p1q0, p1q1, p1q2, p1q3, xr1 = pl.pallas_call(
        _tc1,
        grid=grid,
        in_specs=[pl.BlockSpec((BM, D), rows),
                  pl.BlockSpec((D, H), full),
                  pl.BlockSpec((D, H), full)],
        out_specs=[pl.BlockSpec((BM, QW), rows),
                   pl.BlockSpec((BM, QW), rows),
                   pl.BlockSpec((BM, QW), rows),
                   pl.BlockSpec((BM, QW), rows),
                   pl.BlockSpec((BM, H), rows)],
        out_shape=[jax.ShapeDtypeStruct((N, QW), _F32),
                   jax.ShapeDtypeStruct((N, QW), _F32),
                   jax.ShapeDtypeStruct((N, QW), _F32),
                   jax.ShapeDtypeStruct((N, QW), _F32),
                   jax.ShapeDtypeStruct((N, H), _F32)],
    )(x, W_l1, W_r1)

    mesh = plsc.VectorSubcoreMesh(core_axis_name="c", subcore_axis_name="s")
    sc_params = pltpu.CompilerParams(use_tc_tiling_on_sc=False)
    agg1, deg8 = pl.kernel(
        _sc_l1,
        compiler_params=sc_params,
        out_type=(jax.ShapeDtypeStruct((4 * N, QW), _F32),
                  jax.ShapeDtypeStruct((N, DW), _F32)),
        mesh=mesh,
        scratch_types=(
            pltpu.VMEM((K1, B), jnp.int32),
            pltpu.VMEM((K1, B), jnp.int32),
            pltpu.VMEM((B, QW), _F32),
            pltpu.VMEM((B, QW), _F32),
            pltpu.VMEM((B, QW), _F32),
            pltpu.VMEM((B, QW), _F32),
            pltpu.VMEM((B, DW), _F32),
            pltpu.VMEM((B, DW), _F32),
            pltpu.VMEM_SHARED((N, QW), _F32),
            pltpu.VMEM_SHARED((N, DW), _F32),
            pltpu.SemaphoreType.DMA,
            pltpu.SemaphoreType.DMA,
            pltpu.SemaphoreType.DMA,
            pltpu.SemaphoreType.DMA,
            pltpu.SemaphoreType.DMA,
            pltpu.SemaphoreType.DMA,
            pltpu.SemaphoreType.DMA,
            pltpu.SemaphoreType.DMA,
            pltpu.SemaphoreType.DMA,
        ),
    )(p1q0, p1q1, p1q2, p1q3, ei)

    qrows = [lambda i, q=q: (q * (N // BM) + i, 0) for q in range(4)]
    p2, hr2 = pl.pallas_call(
        _tc2,
        grid=grid,
        in_specs=[pl.BlockSpec((BM, QW), qrows[0]),
                  pl.BlockSpec((BM, QW), qrows[1]),
                  pl.BlockSpec((BM, QW), qrows[2]),
                  pl.BlockSpec((BM, QW), qrows[3]),
                  pl.BlockSpec((BM, DW), rows),
                  pl.BlockSpec((BM, H), rows),
                  pl.BlockSpec((1, H), full),
                  pl.BlockSpec((H, C), full),
                  pl.BlockSpec((H, C), full),
                  pl.BlockSpec((1, C), full)],
        out_specs=[pl.BlockSpec((BM, C), rows),
                   pl.BlockSpec((BM, C), rows)],
        out_shape=[jax.ShapeDtypeStruct((N, C), _F32),
                   jax.ShapeDtypeStruct((N, C), _F32)],
    )(agg1, agg1, agg1, agg1, deg8, xr1, b1r, W_l2, W_r2, b2r)

    agg2 = pl.kernel(
        _sc_l2,
        compiler_params=sc_params,
        out_type=jax.ShapeDtypeStruct((2 * N, C), _F32),
        mesh=mesh,
        scratch_types=(
            pltpu.VMEM((K2, B), jnp.int32),
            pltpu.VMEM((K2, B), jnp.int32),
            pltpu.VMEM((B, C), _F32),
            pltpu.VMEM((B, C), _F32),
            pltpu.VMEM((B, C), _F32),
            pltpu.VMEM((B, C), _F32),
            pltpu.VMEM_SHARED((N, C), _F32),
            pltpu.SemaphoreType.DMA,
            pltpu.SemaphoreType.DMA,
            pltpu.SemaphoreType.DMA,
            pltpu.SemaphoreType.DMA,
            pltpu.SemaphoreType.DMA,
            pltpu.SemaphoreType.DMA,
            pltpu.SemaphoreType.DMA,
            pltpu.SemaphoreType.DMA,
        ),
    )(p2, ei)

    out = pl.pallas_call(
        _tc3,
        grid=grid,
        in_specs=[pl.BlockSpec((BM, C), rows),
                  pl.BlockSpec((BM, C), rows_hi),
                  pl.BlockSpec((BM, DW), rows),
                  pl.BlockSpec((BM, C), rows)],
        out_specs=pl.BlockSpec((BM, C), rows),
        out_shape=jax.ShapeDtypeStruct((N, C), _F32),
    )(agg2, agg2, deg8, hr2)

    return out


# speedup vs baseline: 10.5549x; 1.1860x over previous
import jax
import jax.numpy as jnp
from jax import lax
from jax.experimental import pallas as pl
from jax.experimental.pallas import tpu as pltpu
from jax.experimental.pallas import tpu_sc as plsc

N = 10000
E = 160000
D = 256
H = 256
C = 64

NC = 2
NS = 16
QW = H // 4
DW = 16

BM = 1000
RPT = 624
TAIL = N - NS * RPT
B = 125
K1 = (E // NS) // B
K2 = (E // (NC * NS)) // B

_F32 = jnp.float32


def _fill(ref, rows, cols, value):
    def row(i, carry):
        def col(k, carry2):
            ref[i, pl.ds(k * 16, 16)] = jnp.full((16,), value, _F32)
            return carry2
        lax.fori_loop(0, cols // 16, col, 0)
        return carry
    lax.fori_loop(0, rows, row, 0)


def _zero_slices(zbuf, brows, acc, base):
    nfull = RPT // brows
    rem = RPT - nfull * brows
    def blk(k, carry):
        pltpu.sync_copy(zbuf, acc.at[pl.ds(base + k * brows, brows)])
        return carry
    lax.fori_loop(0, nfull, blk, 0)
    if rem:
        pltpu.sync_copy(zbuf.at[pl.ds(0, rem)],
                        acc.at[pl.ds(base + nfull * brows, rem)])


def _edge_loop(p_ref, src_v, dst_v, acc_s, K, bufs, gsems, ssems, deg=None):
    nb = 4
    pltpu.async_copy(p_ref.at[src_v.at[0]], bufs[0], gsems[0])
    pltpu.async_copy(p_ref.at[src_v.at[1]], bufs[1], gsems[1])

    def step(t, carry):
        j0 = t * nb
        for b in range(nb):
            j = j0 + b
            bn = (b + 2) % nb
            nxt = j + 2

            @pl.when(nxt < K)
            def _():
                @pl.when(nxt >= nb)
                def _():
                    pltpu.make_async_copy(bufs[bn], acc_s.at[dst_v.at[j]],
                                          ssems[bn]).wait()
                pltpu.async_copy(p_ref.at[src_v.at[nxt]], bufs[bn], gsems[bn])

            pltpu.make_async_copy(p_ref.at[src_v.at[j]], bufs[b],
                                  gsems[b]).wait()
            pltpu.async_copy(bufs[b], acc_s.at[dst_v.at[j]], ssems[b],
                             add=True)
            if deg is not None:
                ones_v, deg_s, dsem = deg
                pltpu.async_copy(ones_v, deg_s.at[dst_v.at[j]], dsem, add=True)
        return carry
    lax.fori_loop(0, K // nb, step, 0)

    for b in range(nb):
        pltpu.make_async_copy(bufs[b], acc_s.at[dst_v.at[0]], ssems[b]).wait()
    if deg is not None:
        ones_v, deg_s, dsem = deg

        def drain(j, carry):
            pltpu.make_async_copy(ones_v, deg_s.at[dst_v.at[0]], dsem).wait()
            return carry
        lax.fori_loop(0, K, drain, 0)


def _sc_l1(halfa, halfb, ei,
           agga_out, aggb_out, deg_out,
           src0_v, src1_v, dst_v, bf0, bf1, bf2, bf3, ones_v, zdeg_v,
           acc_s, deg_s,
           g0, g1, g2, g3, s0, s1, s2, s3, dsem):
    c = lax.axis_index("c")
    s = lax.axis_index("s")
    bufs = (bf0, bf1, bf2, bf3)
    gsems = (g0, g1, g2, g3)
    ssems = (s0, s1, s2, s3)

    @pl.when(c == 0)
    def _():
        _fill(ones_v, B, DW, 1.0)
        _fill(zdeg_v, B, DW, 0.0)
        _zero_slices(zdeg_v, B, deg_s, s * RPT)

    @pl.when(jnp.logical_and(c == 0, s == 0))
    def _():
        pltpu.sync_copy(zdeg_v.at[pl.ds(0, TAIL)],
                        deg_s.at[pl.ds(NS * RPT, TAIL)])

    pltpu.sync_copy(ei.at[0, pl.ds(s * K1, K1)], src0_v)
    pltpu.sync_copy(ei.at[1, pl.ds(s * K1, K1)], src1_v)
    pltpu.sync_copy(ei.at[2, pl.ds(s * K1, K1)], dst_v)

    def one_pass(p_ref, agg_out, sv, q, add_deg):
        _fill(bf0, B, QW, 0.0)
        _zero_slices(bf0, B, acc_s, s * RPT)

        @pl.when(s == 0)
        def _():
            pltpu.sync_copy(bf0.at[pl.ds(0, TAIL)],
                            acc_s.at[pl.ds(NS * RPT, TAIL)])

        plsc.subcore_barrier()
        _edge_loop(p_ref, sv, dst_v, acc_s, K1, bufs, gsems, ssems,
                   deg=(ones_v, deg_s, dsem) if add_deg else None)

        plsc.subcore_barrier()
        pltpu.sync_copy(acc_s.at[pl.ds(s * RPT, RPT)],
                        agg_out.at[pl.ds(s * RPT, RPT), pl.ds(q * QW, QW)])

        @pl.when(s == 0)
        def _():
            pltpu.sync_copy(
                acc_s.at[pl.ds(NS * RPT, TAIL)],
                agg_out.at[pl.ds(NS * RPT, TAIL), pl.ds(q * QW, QW)])

        plsc.subcore_barrier()

    @pl.when(c == 0)
    def _():
        one_pass(halfa, agga_out, src0_v, 0, True)
        one_pass(halfa, agga_out, src1_v, 1, False)

    @pl.when(c == 1)
    def _():
        one_pass(halfb, aggb_out, src0_v, 0, False)
        one_pass(halfb, aggb_out, src1_v, 1, False)

    @pl.when(c == 0)
    def _():
        pltpu.sync_copy(deg_s.at[pl.ds(s * RPT, RPT)],
                        deg_out.at[pl.ds(s * RPT, RPT)])

    @pl.when(jnp.logical_and(c == 0, s == 0))
    def _():
        pltpu.sync_copy(deg_s.at[pl.ds(NS * RPT, TAIL)],
                        deg_out.at[pl.ds(NS * RPT, TAIL)])


def _sc_l2(p2, ei,
           agg_out,
           src0_v, dst_v, bf0, bf1, bf2, bf3, acc_s,
           g0, g1, g2, g3, s0, s1, s2, s3):
    c = lax.axis_index("c")
    s = lax.axis_index("s")
    wid = s * NC + c
    bufs = (bf0, bf1, bf2, bf3)
    _fill(bf0, B, C, 0.0)
    _zero_slices(bf0, B, acc_s, s * RPT)

    @pl.when(s == 0)
    def _():
        pltpu.sync_copy(bf0.at[pl.ds(0, TAIL)],
                        acc_s.at[pl.ds(NS * RPT, TAIL)])

    pltpu.sync_copy(ei.at[0, pl.ds(wid * K2, K2)], src0_v)
    pltpu.sync_copy(ei.at[2, pl.ds(wid * K2, K2)], dst_v)
    plsc.subcore_barrier()
    _edge_loop(p2, src0_v, dst_v, acc_s, K2, bufs,
               (g0, g1, g2, g3), (s0, s1, s2, s3))

    plsc.subcore_barrier()

    def writeback(col):
        pltpu.sync_copy(acc_s.at[pl.ds(s * RPT, RPT)],
                        agg_out.at[pl.ds(s * RPT, RPT), pl.ds(col, C)])

        @pl.when(s == 0)
        def _():
            pltpu.sync_copy(acc_s.at[pl.ds(NS * RPT, TAIL)],
                            agg_out.at[pl.ds(NS * RPT, TAIL), pl.ds(col, C)])

    @pl.when(c == 0)
    def _():
        writeback(0)

    @pl.when(c == 1)
    def _():
        writeback(C)


def _tc1(x_ref, wl_ref, wr_ref, ha_ref, hb_ref, xr_ref):
    xb = x_ref[...]
    p = jnp.dot(xb, wl_ref[...], preferred_element_type=_F32)
    ha_ref[...] = p[:, :128]
    hb_ref[...] = p[:, 128:]
    xr_ref[...] = jnp.dot(xb, wr_ref[...], preferred_element_type=_F32)


def _tc2(a_ref, b_ref, deg_ref, xr_ref, b1_ref, wl2_ref, wr2_ref,
         b2_ref, ph_ref):
    agg = jnp.concatenate([a_ref[...], b_ref[...]], axis=1)
    deg = jnp.maximum(deg_ref[...][:, 0:1], 1.0)
    h = jnp.maximum(agg / deg + xr_ref[...] + b1_ref[...], 0.0)
    p2 = jnp.dot(h, wl2_ref[...], preferred_element_type=_F32)
    hr = jnp.dot(h, wr2_ref[...], preferred_element_type=_F32) + b2_ref[...]
    ph_ref[...] = jnp.concatenate([p2, hr], axis=1)


def _tc3(a2_ref, ph_ref, deg_ref, out_ref):
    deg = jnp.maximum(deg_ref[...][:, 0:1], 1.0)
    a2 = a2_ref[...]
    v = (a2[:, :C] + a2[:, C:]) / deg + ph_ref[...][:, C:]
    m = jnp.max(v, axis=1, keepdims=True)
    z = v - m
    lse = jnp.log(jnp.sum(jnp.exp(z), axis=1, keepdims=True))
    out_ref[...] = z - lse


def kernel(x, G2_edge_attr, G1_edge_attr_matrix, G3_edge_index, G3_edge_attr,
           W_l1, W_r1, b1, W_l2, W_r2, b2):
    src = G3_edge_index[0]
    dst = G3_edge_index[1]
    ei = jnp.stack([src * 2, src * 2 + 1, dst]).reshape(3, E // B, B)
    b1r = b1.reshape(1, H)
    b2r = b2.reshape(1, C)

    grid = (N // BM,)
    full = lambda i: (0, 0)
    rows = lambda i: (i, 0)

    halfa, halfb, xr1 = pl.pallas_call(
        _tc1,
        grid=grid,
        in_specs=[pl.BlockSpec((BM, D), rows),
                  pl.BlockSpec((D, H), full),
                  pl.BlockSpec((D, H), full)],
        out_specs=[pl.BlockSpec((BM, 128), rows),
                   pl.BlockSpec((BM, 128), rows),
                   pl.BlockSpec((BM, H), rows)],
        out_shape=[jax.ShapeDtypeStruct((N, 128), _F32),
                   jax.ShapeDtypeStruct((N, 128), _F32),
                   jax.ShapeDtypeStruct((N, H), _F32)],
    )(x, W_l1, W_r1)

    mesh = plsc.VectorSubcoreMesh(core_axis_name="c", subcore_axis_name="s")
    sc_params = pltpu.CompilerParams(use_tc_tiling_on_sc=False)
    agga, aggb, deg8 = pl.kernel(
        _sc_l1,
        compiler_params=sc_params,
        out_type=(jax.ShapeDtypeStruct((N, 128), _F32),
                  jax.ShapeDtypeStruct((N, 128), _F32),
                  jax.ShapeDtypeStruct((N, DW), _F32)),
        mesh=mesh,
        scratch_types=(
            pltpu.VMEM((K1, B), jnp.int32),
            pltpu.VMEM((K1, B), jnp.int32),
            pltpu.VMEM((K1, B), jnp.int32),
            pltpu.VMEM((B, QW), _F32),
            pltpu.VMEM((B, QW), _F32),
            pltpu.VMEM((B, QW), _F32),
            pltpu.VMEM((B, QW), _F32),
            pltpu.VMEM((B, DW), _F32),
            pltpu.VMEM((B, DW), _F32),
            pltpu.VMEM_SHARED((N, QW), _F32),
            pltpu.VMEM_SHARED((N, DW), _F32),
            pltpu.SemaphoreType.DMA,
            pltpu.SemaphoreType.DMA,
            pltpu.SemaphoreType.DMA,
            pltpu.SemaphoreType.DMA,
            pltpu.SemaphoreType.DMA,
            pltpu.SemaphoreType.DMA,
            pltpu.SemaphoreType.DMA,
            pltpu.SemaphoreType.DMA,
            pltpu.SemaphoreType.DMA,
        ),
    )(halfa.reshape(2 * N, QW), halfb.reshape(2 * N, QW), ei)

    p2hr = pl.pallas_call(
        _tc2,
        grid=grid,
        in_specs=[pl.BlockSpec((BM, 128), rows),
                  pl.BlockSpec((BM, 128), rows),
                  pl.BlockSpec((BM, DW), rows),
                  pl.BlockSpec((BM, H), rows),
                  pl.BlockSpec((1, H), full),
                  pl.BlockSpec((H, C), full),
                  pl.BlockSpec((H, C), full),
                  pl.BlockSpec((1, C), full)],
        out_specs=pl.BlockSpec((BM, 128), rows),
        out_shape=jax.ShapeDtypeStruct((N, 128), _F32),
    )(agga, aggb, deg8, xr1, b1r, W_l2, W_r2, b2r)

    agg2 = pl.kernel(
        _sc_l2,
        compiler_params=sc_params,
        out_type=jax.ShapeDtypeStruct((N, 128), _F32),
        mesh=mesh,
        scratch_types=(
            pltpu.VMEM((K2, B), jnp.int32),
            pltpu.VMEM((K2, B), jnp.int32),
            pltpu.VMEM((B, C), _F32),
            pltpu.VMEM((B, C), _F32),
            pltpu.VMEM((B, C), _F32),
            pltpu.VMEM((B, C), _F32),
            pltpu.VMEM_SHARED((N, C), _F32),
            pltpu.SemaphoreType.DMA,
            pltpu.SemaphoreType.DMA,
            pltpu.SemaphoreType.DMA,
            pltpu.SemaphoreType.DMA,
            pltpu.SemaphoreType.DMA,
            pltpu.SemaphoreType.DMA,
            pltpu.SemaphoreType.DMA,
            pltpu.SemaphoreType.DMA,
        ),
    )(p2hr.reshape(2 * N, C), ei)

    out = pl.pallas_call(
        _tc3,
        grid=grid,
        in_specs=[pl.BlockSpec((BM, 128), rows),
                  pl.BlockSpec((BM, 128), rows),
                  pl.BlockSpec((BM, DW), rows)],
        out_specs=pl.BlockSpec((BM, C), rows),
        out_shape=jax.ShapeDtypeStruct((N, C), _F32),
    )(agg2, p2hr, deg8)

    return out
